# Initial kernel scaffold; baseline (speedup 1.0000x reference)
import jax
import jax.numpy as jnp
from jax.experimental import pallas as pl


def kernel(inputs, embed_table, W, b):
    raise NotImplementedError("write your pallas kernel here")


# speedup vs baseline: 2.3782x; 2.3782x over previous
import functools

import jax
import jax.numpy as jnp
from jax import lax
from jax.experimental import pallas as pl
from jax.experimental.pallas import tpu as pltpu
from jax.experimental.pallas import tpu_sc as plsc

VOCAB = 1000000
EMBED = 64
NCLS = 5
BATCH = 16384
HIST = 200

DP = 16
NC, NS = 2, 16
NW = NC * NS
SPW = BATCH // NW
CS = 8
NCHUNK = SPW // CS
IPG = HIST // 2
KPC = 2 * CS


def _proj_body(e_ref, w_ref, p_ref):
    p_ref[...] = jnp.dot(e_ref[...], w_ref[...],
                         preferred_element_type=jnp.float32)


def _project_table(embed_table, w_pad):
    rb = 8000
    return pl.pallas_call(
        _proj_body,
        grid=(VOCAB // rb,),
        in_specs=[
            pl.BlockSpec((rb, EMBED), lambda i: (i, 0)),
            pl.BlockSpec((EMBED, DP), lambda i: (0, 0)),
        ],
        out_specs=pl.BlockSpec((rb, DP), lambda i: (i, 0)),
        out_shape=jax.ShapeDtypeStruct((VOCAB, DP), jnp.float32),
    )(embed_table, w_pad)


def _sc_body(p_hbm, idx_hbm, b_hbm, out_hbm, idxb, rowsb, outb, bvecb, sem_g):
    wid = lax.axis_index("s") * NC + lax.axis_index("c")
    base = wid * SPW
    pltpu.sync_copy(b_hbm, bvecb)

    def chunk(c, carry):
        srow = base + c * CS
        pltpu.sync_copy(idx_hbm.at[pl.ds(srow * 2, KPC)], idxb)
        cps = [pltpu.async_copy(p_hbm.at[idxb.at[k]], rowsb.at[k], sem_g)
               for k in range(KPC)]
        for cp in cps:
            cp.wait()
        bv = bvecb[...]
        z = jnp.zeros((DP,), jnp.float32)
        for s in range(CS):
            def body(j, accs):
                a0, a1, a2, a3 = accs
                return (a0 + rowsb[2 * s, 2 * j],
                        a1 + rowsb[2 * s, 2 * j + 1],
                        a2 + rowsb[2 * s + 1, 2 * j],
                        a3 + rowsb[2 * s + 1, 2 * j + 1])
            a0, a1, a2, a3 = lax.fori_loop(0, IPG // 2, body, (z, z, z, z))
            outb[s] = (a0 + a1) + (a2 + a3) + bv
        pltpu.sync_copy(outb, out_hbm.at[pl.ds(srow, CS)])
        return carry

    lax.fori_loop(0, NCHUNK, chunk, 0)


@jax.jit
def kernel(inputs, embed_table, W, b):
    w_pad = jnp.zeros((EMBED, DP), jnp.float32).at[:, :NCLS].set(W)
    b_pad = jnp.zeros((DP,), jnp.float32).at[:NCLS].set(b)
    p = _project_table(embed_table, w_pad)
    idx2 = jnp.reshape(inputs.astype(jnp.int32), (BATCH * 2, IPG))

    mesh = plsc.VectorSubcoreMesh(core_axis_name="c", subcore_axis_name="s")
    out16 = pl.kernel(
        _sc_body,
        out_type=jax.ShapeDtypeStruct((BATCH, DP), jnp.float32),
        mesh=mesh,
        compiler_params=pltpu.CompilerParams(use_tc_tiling_on_sc=False),
        scratch_types=[
            pltpu.VMEM((KPC, IPG), jnp.int32),
            pltpu.VMEM((KPC, IPG, DP), jnp.float32),
            pltpu.VMEM((CS, DP), jnp.float32),
            pltpu.VMEM((DP,), jnp.float32),
            pltpu.SemaphoreType.DMA,
        ],
    )(p, idx2, b_pad)
    return out16[:, :NCLS]


# speedup vs baseline: 5.3613x; 2.2544x over previous
import functools

import jax
import jax.numpy as jnp
from jax import lax
from jax.experimental import pallas as pl
from jax.experimental.pallas import tpu as pltpu
from jax.experimental.pallas import tpu_sc as plsc

VOCAB = 1000000
EMBED = 64
NCLS = 5
BATCH = 16384
HIST = 200

DP = 16
NC, NS = 2, 16
NW = NC * NS
SPW = BATCH // NW
CS = 8
NCHUNK = SPW // CS
IPG = HIST // 2
KPC = 2 * CS


RB = 8192
NBLK = -(-VOCAB // RB)
VP = NBLK * RB


def _proj_body(et_ref, w_ref, p_ref):
    acc = lax.dot_general(et_ref[:, 0:1024], w_ref[:, 0:128],
                          dimension_numbers=(((0,), (0,)), ((), ())),
                          preferred_element_type=jnp.float32)
    for m in range(1, 8):
        acc += lax.dot_general(et_ref[:, 1024 * m:1024 * (m + 1)],
                               w_ref[:, 128 * m:128 * (m + 1)],
                               dimension_numbers=(((0,), (0,)), ((), ())),
                               preferred_element_type=jnp.float32)
    p_ref[...] = acc


def _project_table(embed_t, w_pad):
    return pl.pallas_call(
        _proj_body,
        grid=(NBLK,),
        in_specs=[
            pl.BlockSpec((EMBED, RB), lambda i: (0, i)),
            pl.BlockSpec((EMBED, 1024), lambda i: (0, 0)),
        ],
        out_specs=pl.BlockSpec((RB // 8, 128), lambda i: (i, 0)),
        out_shape=jax.ShapeDtypeStruct((VP // 8, 128), jnp.float32),
    )(embed_t, w_pad)


def _sc_body(p_hbm, idx_hbm, b_hbm, out_hbm, idxb, rowsb, outb, bvecb, sem_g):
    wid = lax.axis_index("s") * NC + lax.axis_index("c")
    base = wid * SPW
    pltpu.sync_copy(b_hbm, bvecb)

    def chunk(c, carry):
        srow = base + c * CS
        pltpu.sync_copy(idx_hbm.at[pl.ds(srow * 2, KPC)], idxb)
        cps = [pltpu.async_copy(p_hbm.at[idxb.at[k]], rowsb.at[k], sem_g)
               for k in range(KPC)]
        for cp in cps:
            cp.wait()
        bv = bvecb[...]
        z = jnp.zeros((DP,), jnp.float32)
        for s in range(CS):
            def body(j, accs):
                a0, a1, a2, a3 = accs
                return (a0 + rowsb[2 * s, 2 * j],
                        a1 + rowsb[2 * s, 2 * j + 1],
                        a2 + rowsb[2 * s + 1, 2 * j],
                        a3 + rowsb[2 * s + 1, 2 * j + 1])
            a0, a1, a2, a3 = lax.fori_loop(0, IPG // 2, body, (z, z, z, z))
            outb[s] = (a0 + a1) + (a2 + a3) + bv
        pltpu.sync_copy(outb, out_hbm.at[pl.ds(srow, CS)])
        return carry

    lax.fori_loop(0, NCHUNK, chunk, 0)


@jax.jit
def kernel(inputs, embed_table, W, b):
    b_pad = jnp.zeros((DP,), jnp.float32).at[:NCLS].set(b)
    w_big = jnp.zeros((EMBED, 8 * 128), jnp.float32)
    for _m in range(8):
        w_big = w_big.at[:, 128 * _m + 16 * _m:128 * _m + 16 * _m + NCLS].set(W)
    p = jnp.reshape(_project_table(embed_table.T, w_big), (VP, DP))
    v = inputs.astype(jnp.int32)
    vg = (v & ~(RB - 1)) | ((v & 1023) << 3) | ((v >> 10) & 7)
    idx2 = jnp.reshape(vg, (BATCH * 2, IPG))

    mesh = plsc.VectorSubcoreMesh(core_axis_name="c", subcore_axis_name="s")
    out16 = pl.kernel(
        _sc_body,
        out_type=jax.ShapeDtypeStruct((BATCH, DP), jnp.float32),
        mesh=mesh,
        compiler_params=pltpu.CompilerParams(use_tc_tiling_on_sc=False),
        scratch_types=[
            pltpu.VMEM((KPC, IPG), jnp.int32),
            pltpu.VMEM((KPC, IPG, DP), jnp.float32),
            pltpu.VMEM((CS, DP), jnp.float32),
            pltpu.VMEM((DP,), jnp.float32),
            pltpu.SemaphoreType.DMA,
        ],
    )(p, idx2, b_pad)
    return out16[:, :NCLS]


# speedup vs baseline: 7.1943x; 1.3419x over previous
import functools

import jax
import jax.numpy as jnp
from jax import lax
from jax.experimental import pallas as pl
from jax.experimental.pallas import tpu as pltpu
from jax.experimental.pallas import tpu_sc as plsc

VOCAB = 1000000
EMBED = 64
NCLS = 5
BATCH = 16384
HIST = 200

DP = 16
NC, NS = 2, 16
NW = NC * NS
SPW = BATCH // NW
CS = 8
NCHUNK = SPW // CS
G0, G1 = 104, 96


RB = 8192
NBLK = -(-VOCAB // RB)
VP = NBLK * RB


def _proj_body(et_ref, w_ref, p_ref):
    acc = lax.dot_general(et_ref[:, 0:1024], w_ref[:, 0:128],
                          dimension_numbers=(((0,), (0,)), ((), ())),
                          preferred_element_type=jnp.float32)
    for m in range(1, 8):
        acc += lax.dot_general(et_ref[:, 1024 * m:1024 * (m + 1)],
                               w_ref[:, 128 * m:128 * (m + 1)],
                               dimension_numbers=(((0,), (0,)), ((), ())),
                               preferred_element_type=jnp.float32)
    p_ref[...] = acc


def _project_table(embed_t, w_pad):
    return pl.pallas_call(
        _proj_body,
        grid=(NBLK,),
        in_specs=[
            pl.BlockSpec((EMBED, RB), lambda i: (0, i)),
            pl.BlockSpec((EMBED, 1024), lambda i: (0, 0)),
        ],
        out_specs=pl.BlockSpec((RB // 8, 128), lambda i: (i, 0)),
        out_shape=jax.ShapeDtypeStruct((VP // 8, 128), jnp.float32),
    )(embed_t, w_pad)


def _sc_body(p_hbm, idx_hbm, b_hbm, out_hbm,
             idxb0, idxb1, rowsb0, rowsb1, outv, bvecb,
             semg0, semg1, semi0, semi1):
    wid = lax.axis_index("s") * NC + lax.axis_index("c")
    base = wid * SPW
    pltpu.sync_copy(b_hbm, bvecb)
    bufs = ((idxb0, rowsb0, semg0, semi0), (idxb1, rowsb1, semg1, semi1))

    def stage_idx(c, idxb, semi):
        pltpu.async_copy(idx_hbm.at[pl.ds(base + c * CS, CS)], idxb, semi)

    def wait_idx(c, idxb, semi):
        pltpu.make_async_copy(
            idx_hbm.at[pl.ds(base + c * CS, CS)], idxb, semi).wait()

    def fire_gathers(idxb, rowsb, semg):
        for s in range(CS):
            pltpu.async_copy(p_hbm.at[idxb.at[s, pl.ds(0, G0)]],
                             rowsb.at[pl.ds(HIST * s, G0)], semg)
            pltpu.async_copy(p_hbm.at[idxb.at[s, pl.ds(G0, G1)]],
                             rowsb.at[pl.ds(HIST * s + G0, G1)], semg)

    def drain_gathers(rowsb, semg):
        pltpu.make_async_copy(p_hbm.at[pl.ds(0, CS * HIST)], rowsb, semg).wait()

    def accumulate(c, rowsb):
        bv = bvecb[...]
        z = jnp.zeros((DP,), jnp.float32)
        for s in range(CS):
            def body(j, accs):
                r = HIST * s + 8 * j
                return tuple(accs[t] + rowsb[r + t] for t in range(8))
            accs = lax.fori_loop(0, HIST // 8, body, (z,) * 8)
            tot = (((accs[0] + accs[1]) + (accs[2] + accs[3]))
                   + ((accs[4] + accs[5]) + (accs[6] + accs[7]))) + bv
            outv[c * CS + s] = tot

    idxb, rowsb, semg, semi = bufs[0]
    stage_idx(0, idxb, semi)
    wait_idx(0, idxb, semi)
    fire_gathers(idxb, rowsb, semg)
    stage_idx(1, bufs[1][0], bufs[1][3])

    def half(c, b):
        idxb, rowsb, semg, semi = bufs[b]
        nidxb, nrowsb, nsemg, nsemi = bufs[1 - b]
        drain_gathers(rowsb, semg)

        @pl.when(c + 2 < NCHUNK)
        def _():
            stage_idx(c + 2, idxb, semi)

        @pl.when(c + 1 < NCHUNK)
        def _():
            wait_idx(c + 1, nidxb, nsemi)
            fire_gathers(nidxb, nrowsb, nsemg)

        accumulate(c, rowsb)

    def pair(cc, carry):
        half(2 * cc, 0)
        half(2 * cc + 1, 1)
        return carry

    lax.fori_loop(0, NCHUNK // 2, pair, 0)
    pltpu.sync_copy(outv, out_hbm.at[pl.ds(base, SPW)])


@jax.jit
def kernel(inputs, embed_table, W, b):
    b_pad = jnp.zeros((DP,), jnp.float32).at[:NCLS].set(b)
    w_big = jnp.zeros((EMBED, 8 * 128), jnp.float32)
    for _m in range(8):
        w_big = w_big.at[:, 128 * _m + 16 * _m:128 * _m + 16 * _m + NCLS].set(W)
    p = jnp.reshape(_project_table(embed_table.T, w_big), (VP, DP))
    v = inputs.astype(jnp.int32)
    vg = (v & ~(RB - 1)) | ((v & 1023) << 3) | ((v >> 10) & 7)

    mesh = plsc.VectorSubcoreMesh(core_axis_name="c", subcore_axis_name="s")
    out16 = pl.kernel(
        _sc_body,
        out_type=jax.ShapeDtypeStruct((BATCH, DP), jnp.float32),
        mesh=mesh,
        compiler_params=pltpu.CompilerParams(use_tc_tiling_on_sc=False),
        scratch_types=[
            pltpu.VMEM((CS, HIST), jnp.int32),
            pltpu.VMEM((CS, HIST), jnp.int32),
            pltpu.VMEM((CS * HIST, DP), jnp.float32),
            pltpu.VMEM((CS * HIST, DP), jnp.float32),
            pltpu.VMEM((SPW, DP), jnp.float32),
            pltpu.VMEM((DP,), jnp.float32),
            pltpu.SemaphoreType.DMA,
            pltpu.SemaphoreType.DMA,
            pltpu.SemaphoreType.DMA,
            pltpu.SemaphoreType.DMA,
        ],
    )(p, vg, b_pad)
    return out16[:, :NCLS]


# speedup vs baseline: 8.3969x; 1.1672x over previous
import functools

import jax
import jax.numpy as jnp
from jax import lax
from jax.experimental import pallas as pl
from jax.experimental.pallas import tpu as pltpu
from jax.experimental.pallas import tpu_sc as plsc

VOCAB = 1000000
EMBED = 64
NCLS = 5
BATCH = 16384
HIST = 200

DP = 16
NC, NS = 2, 16
NW = NC * NS
SPW = BATCH // NW
CS = 16
NCHUNK = SPW // CS
G0, G1 = 104, 96


RB = 8192
NBLK = -(-VOCAB // RB)
VP = NBLK * RB


def _proj_body(et_ref, w_ref, p_ref):
    lhs = jnp.concatenate(
        [et_ref[:, 1024 * m:1024 * (m + 1)] for m in range(8)], axis=0)
    p_ref[...] = lax.dot_general(lhs, w_ref[...],
                                 dimension_numbers=(((0,), (0,)), ((), ())),
                                 preferred_element_type=jnp.float32)


def _project_table(embed_t, w_pad):
    return pl.pallas_call(
        _proj_body,
        grid=(NBLK,),
        in_specs=[
            pl.BlockSpec((EMBED, RB), lambda i: (0, i)),
            pl.BlockSpec((8 * EMBED, 128), lambda i: (0, 0)),
        ],
        out_specs=pl.BlockSpec((RB // 8, 128), lambda i: (i, 0)),
        out_shape=jax.ShapeDtypeStruct((VP // 8, 128), jnp.float32),
    )(embed_t, w_pad)


def _sc_body(p_hbm, idx_hbm, b_hbm, out_hbm,
             idxb0, idxb1, rowsb0, rowsb1, outv, bvecb,
             semg0, semg1, semi0, semi1):
    wid = lax.axis_index("s") * NC + lax.axis_index("c")
    base = wid * SPW
    pltpu.sync_copy(b_hbm, bvecb)
    bufs = ((idxb0, rowsb0, semg0, semi0), (idxb1, rowsb1, semg1, semi1))

    def stage_idx(c, idxb, semi):
        pltpu.async_copy(idx_hbm.at[pl.ds(base + c * CS, CS)], idxb, semi)

    def wait_idx(c, idxb, semi):
        pltpu.make_async_copy(
            idx_hbm.at[pl.ds(base + c * CS, CS)], idxb, semi).wait()

    def fire_gathers(idxb, rowsb, semg):
        for s in range(CS):
            pltpu.async_copy(p_hbm.at[idxb.at[s, pl.ds(0, G0)]],
                             rowsb.at[pl.ds(HIST * s, G0)], semg)
            pltpu.async_copy(p_hbm.at[idxb.at[s, pl.ds(G0, G1)]],
                             rowsb.at[pl.ds(HIST * s + G0, G1)], semg)

    def drain_gathers(rowsb, semg):
        pltpu.make_async_copy(p_hbm.at[pl.ds(0, CS * HIST)], rowsb, semg).wait()

    def accumulate(c, rowsb):
        bv = bvecb[...]
        z = jnp.zeros((DP,), jnp.float32)
        for s in range(CS):
            def body(j, accs):
                r = HIST * s + 8 * j
                return tuple(accs[t] + rowsb[r + t] for t in range(8))
            accs = lax.fori_loop(0, HIST // 8, body, (z,) * 8)
            tot = (((accs[0] + accs[1]) + (accs[2] + accs[3]))
                   + ((accs[4] + accs[5]) + (accs[6] + accs[7]))) + bv
            outv[c * CS + s] = tot

    idxb, rowsb, semg, semi = bufs[0]
    stage_idx(0, idxb, semi)
    wait_idx(0, idxb, semi)
    fire_gathers(idxb, rowsb, semg)
    stage_idx(1, bufs[1][0], bufs[1][3])

    def half(c, b):
        idxb, rowsb, semg, semi = bufs[b]
        nidxb, nrowsb, nsemg, nsemi = bufs[1 - b]
        drain_gathers(rowsb, semg)

        @pl.when(c + 2 < NCHUNK)
        def _():
            stage_idx(c + 2, idxb, semi)

        @pl.when(c + 1 < NCHUNK)
        def _():
            wait_idx(c + 1, nidxb, nsemi)
            fire_gathers(nidxb, nrowsb, nsemg)

        accumulate(c, rowsb)

    def pair(cc, carry):
        half(2 * cc, 0)
        half(2 * cc + 1, 1)
        return carry

    lax.fori_loop(0, NCHUNK // 2, pair, 0)
    pltpu.sync_copy(outv, out_hbm.at[pl.ds(base, SPW)])


@jax.jit
def kernel(inputs, embed_table, W, b):
    b_pad = jnp.zeros((DP,), jnp.float32).at[:NCLS].set(b)
    w_bd = jnp.zeros((8 * EMBED, 128), jnp.float32)
    for _m in range(8):
        w_bd = w_bd.at[EMBED * _m:EMBED * (_m + 1),
                       16 * _m:16 * _m + NCLS].set(W)
    p = jnp.reshape(_project_table(embed_table.T, w_bd), (VP, DP))
    v = inputs.astype(jnp.int32)
    vg = (v & ~(RB - 1)) | ((v & 1023) << 3) | ((v >> 10) & 7)

    mesh = plsc.VectorSubcoreMesh(core_axis_name="c", subcore_axis_name="s")
    out16 = pl.kernel(
        _sc_body,
        out_type=jax.ShapeDtypeStruct((BATCH, DP), jnp.float32),
        mesh=mesh,
        compiler_params=pltpu.CompilerParams(use_tc_tiling_on_sc=False),
        scratch_types=[
            pltpu.VMEM((CS, HIST), jnp.int32),
            pltpu.VMEM((CS, HIST), jnp.int32),
            pltpu.VMEM((CS * HIST, DP), jnp.float32),
            pltpu.VMEM((CS * HIST, DP), jnp.float32),
            pltpu.VMEM((SPW, DP), jnp.float32),
            pltpu.VMEM((DP,), jnp.float32),
            pltpu.SemaphoreType.DMA,
            pltpu.SemaphoreType.DMA,
            pltpu.SemaphoreType.DMA,
            pltpu.SemaphoreType.DMA,
        ],
    )(p, vg, b_pad)
    return out16[:, :NCLS]


# speedup vs baseline: 10.2111x; 1.2161x over previous
import functools

import jax
import jax.numpy as jnp
from jax import lax
from jax.experimental import pallas as pl
from jax.experimental.pallas import tpu as pltpu
from jax.experimental.pallas import tpu_sc as plsc

VOCAB = 1000000
EMBED = 64
NCLS = 5
BATCH = 16384
HIST = 200

DP = 16
NC, NS = 2, 16
NW = NC * NS
SPW = BATCH // NW
CS = 16
NCHUNK = SPW // CS
G0, G1 = 104, 96


RB = 32768
SB = RB // 8
SHIFT = 12
NBLK = -(-VOCAB // RB)
VP = NBLK * RB


def _proj_body(et_ref, w_ref, p_ref):
    lhs = jnp.concatenate(
        [et_ref[:, SB * m:SB * (m + 1)] for m in range(8)], axis=0)
    p_ref[...] = lax.dot_general(lhs, w_ref[...],
                                 dimension_numbers=(((0,), (0,)), ((), ())),
                                 preferred_element_type=jnp.float32)


def _project_table(embed_t, w_pad):
    return pl.pallas_call(
        _proj_body,
        grid=(NBLK,),
        in_specs=[
            pl.BlockSpec((EMBED, RB), lambda i: (0, i)),
            pl.BlockSpec((8 * EMBED, 128), lambda i: (0, 0)),
        ],
        out_specs=pl.BlockSpec((RB // 8, 128), lambda i: (i, 0)),
        out_shape=jax.ShapeDtypeStruct((VP // 8, 128), jnp.float32),
    )(embed_t, w_pad)


def _sc_body(p_hbm, idx_hbm, b_hbm, out_hbm,
             idxb0, idxb1, rowsb0, rowsb1, outv, bvecb,
             semg0, semg1, semi0, semi1):
    wid = lax.axis_index("s") * NC + lax.axis_index("c")
    base = wid * SPW
    pltpu.sync_copy(b_hbm, bvecb)
    bufs = ((idxb0, rowsb0, semg0, semi0), (idxb1, rowsb1, semg1, semi1))

    def stage_idx(c, idxb, semi):
        pltpu.async_copy(
            idx_hbm.at[pl.ds((base + c * CS) * HIST, CS * HIST)], idxb, semi)

    def wait_idx(c, idxb, semi):
        pltpu.make_async_copy(
            idx_hbm.at[pl.ds((base + c * CS) * HIST, CS * HIST)],
            idxb, semi).wait()

    def remap(idxb):
        def rbody(j, carry):
            sl = pl.ds(j * 16, 16)
            vv = idxb[sl]
            idxb[sl] = ((vv & jnp.int32(~(RB - 1)))
                        | ((vv & jnp.int32(SB - 1)) << 3)
                        | ((vv >> SHIFT) & 7))
            return carry
        lax.fori_loop(0, CS * HIST // 16, rbody, 0)

    def fire_gathers(idxb, rowsb, semg):
        for s in range(CS):
            pltpu.async_copy(p_hbm.at[idxb.at[pl.ds(HIST * s, G0)]],
                             rowsb.at[pl.ds(HIST * s, G0)], semg)
            pltpu.async_copy(p_hbm.at[idxb.at[pl.ds(HIST * s + G0, G1)]],
                             rowsb.at[pl.ds(HIST * s + G0, G1)], semg)

    def drain_gathers(rowsb, semg):
        pltpu.make_async_copy(p_hbm.at[pl.ds(0, CS * HIST)], rowsb, semg).wait()

    def accumulate(c, rowsb):
        bv = bvecb[...]
        z = jnp.zeros((DP,), jnp.float32)
        for s in range(CS):
            def body(j, accs):
                r = HIST * s + 8 * j
                return tuple(accs[t] + rowsb[r + t] for t in range(8))
            accs = lax.fori_loop(0, HIST // 8, body, (z,) * 8)
            tot = (((accs[0] + accs[1]) + (accs[2] + accs[3]))
                   + ((accs[4] + accs[5]) + (accs[6] + accs[7]))) + bv
            outv[c * CS + s] = tot

    idxb, rowsb, semg, semi = bufs[0]
    stage_idx(0, idxb, semi)
    wait_idx(0, idxb, semi)
    remap(idxb)
    fire_gathers(idxb, rowsb, semg)
    stage_idx(1, bufs[1][0], bufs[1][3])
    wait_idx(1, bufs[1][0], bufs[1][3])
    remap(bufs[1][0])

    def half(c, b):
        idxb, rowsb, semg, semi = bufs[b]
        nidxb, nrowsb, nsemg, nsemi = bufs[1 - b]
        drain_gathers(rowsb, semg)

        @pl.when(c + 1 < NCHUNK)
        def _():
            fire_gathers(nidxb, nrowsb, nsemg)

        @pl.when(c + 2 < NCHUNK)
        def _():
            stage_idx(c + 2, idxb, semi)

        accumulate(c, rowsb)

        @pl.when(c + 2 < NCHUNK)
        def _():
            wait_idx(c + 2, idxb, semi)
            remap(idxb)

    def pair(cc, carry):
        half(2 * cc, 0)
        half(2 * cc + 1, 1)
        return carry

    lax.fori_loop(0, NCHUNK // 2, pair, 0)
    pltpu.sync_copy(outv, out_hbm.at[pl.ds(base, SPW)])


@jax.jit
def kernel(inputs, embed_table, W, b):
    b_pad = jnp.zeros((DP,), jnp.float32).at[:NCLS].set(b)
    w_bd = jnp.zeros((8 * EMBED, 128), jnp.float32)
    for _m in range(8):
        w_bd = w_bd.at[EMBED * _m:EMBED * (_m + 1),
                       16 * _m:16 * _m + NCLS].set(W)
    p = jnp.reshape(_project_table(embed_table.T, w_bd), (VP, DP))
    idx1 = jnp.reshape(inputs.astype(jnp.int32), (BATCH * HIST,))

    mesh = plsc.VectorSubcoreMesh(core_axis_name="c", subcore_axis_name="s")
    out16 = pl.kernel(
        _sc_body,
        out_type=jax.ShapeDtypeStruct((BATCH, DP), jnp.float32),
        mesh=mesh,
        compiler_params=pltpu.CompilerParams(use_tc_tiling_on_sc=False),
        scratch_types=[
            pltpu.VMEM((CS * HIST,), jnp.int32),
            pltpu.VMEM((CS * HIST,), jnp.int32),
            pltpu.VMEM((CS * HIST, DP), jnp.float32),
            pltpu.VMEM((CS * HIST, DP), jnp.float32),
            pltpu.VMEM((SPW, DP), jnp.float32),
            pltpu.VMEM((DP,), jnp.float32),
            pltpu.SemaphoreType.DMA,
            pltpu.SemaphoreType.DMA,
            pltpu.SemaphoreType.DMA,
            pltpu.SemaphoreType.DMA,
        ],
    )(p, idx1, b_pad)
    return out16[:, :NCLS]


# speedup vs baseline: 10.2857x; 1.0073x over previous
import functools

import jax
import jax.numpy as jnp
from jax import lax
from jax.experimental import pallas as pl
from jax.experimental.pallas import tpu as pltpu
from jax.experimental.pallas import tpu_sc as plsc

VOCAB = 1000000
EMBED = 64
NCLS = 5
BATCH = 16384
HIST = 200

DP = 8
NC, NS = 2, 16
NW = NC * NS
SPW = BATCH // NW
CS = 16
NCHUNK = SPW // CS
G0, G1 = 104, 96


RB = 32768
SLOTS = 128 // DP
SB = RB // SLOTS
SHIFT = 11
NBLK = -(-VOCAB // RB)
VP = NBLK * RB


def _proj_body(et_ref, w_ref, p_ref):
    lhs = jnp.concatenate(
        [et_ref[:, SB * m:SB * (m + 1)] for m in range(SLOTS)], axis=0)
    p_ref[...] = lax.dot_general(lhs, w_ref[...],
                                 dimension_numbers=(((0,), (0,)), ((), ())),
                                 preferred_element_type=jnp.float32)


def _project_table(embed_t, w_pad):
    return pl.pallas_call(
        _proj_body,
        grid=(NBLK,),
        in_specs=[
            pl.BlockSpec((EMBED, RB), lambda i: (0, i)),
            pl.BlockSpec((SLOTS * EMBED, 128), lambda i: (0, 0)),
        ],
        out_specs=pl.BlockSpec((RB // SLOTS, 128), lambda i: (i, 0)),
        out_shape=jax.ShapeDtypeStruct((VP // SLOTS, 128), jnp.float32),
    )(embed_t, w_pad)


def _sc_body(p_hbm, idx_hbm, b_hbm, out_hbm,
             idxb0, idxb1, rowsb0, rowsb1, outv, bvecb, foldb,
             semg0, semg1, semi0, semi1):
    wid = lax.axis_index("s") * NC + lax.axis_index("c")
    base = wid * SPW
    pltpu.sync_copy(b_hbm, bvecb)
    foldb[pl.ds(8, 16)] = jnp.zeros((16,), jnp.float32)
    bufs = ((idxb0, rowsb0, semg0, semi0), (idxb1, rowsb1, semg1, semi1))

    def stage_idx(c, idxb, semi):
        pltpu.async_copy(
            idx_hbm.at[pl.ds((base + c * CS) * HIST, CS * HIST)], idxb, semi)

    def wait_idx(c, idxb, semi):
        pltpu.make_async_copy(
            idx_hbm.at[pl.ds((base + c * CS) * HIST, CS * HIST)],
            idxb, semi).wait()

    def remap(idxb):
        def rbody(j, carry):
            sl = pl.ds(j * 16, 16)
            vv = idxb[sl]
            idxb[sl] = ((vv & jnp.int32(~(RB - 1)))
                        | ((vv & jnp.int32(SB - 1)) << 4)
                        | ((vv >> SHIFT) & (SLOTS - 1)))
            return carry
        lax.fori_loop(0, CS * HIST // 16, rbody, 0)

    def fire_gathers(idxb, rowsb, semg):
        for s in range(CS):
            pltpu.async_copy(p_hbm.at[idxb.at[pl.ds(HIST * s, G0)]],
                             rowsb.at[pl.ds(HIST * s, G0)], semg)
            pltpu.async_copy(p_hbm.at[idxb.at[pl.ds(HIST * s + G0, G1)]],
                             rowsb.at[pl.ds(HIST * s + G0, G1)], semg)

    def drain_gathers(rowsb, semg):
        pltpu.make_async_copy(p_hbm.at[pl.ds(0, CS * HIST)], rowsb, semg).wait()

    def accumulate(c, rowsb):
        bv = bvecb[...]
        z = jnp.zeros((16,), jnp.float32)
        rowpat = (lax.iota(jnp.int32, 16) >> 3) & 1
        colpat = lax.iota(jnp.int32, 16) & 7
        for s in range(CS):
            def body(j, accs):
                r = HIST * s + 8 * j
                return tuple(
                    accs[t] + plsc.load_gather(
                        rowsb, [rowpat + (r + 2 * t), colpat])
                    for t in range(4))
            accs = lax.fori_loop(0, HIST // 8, body, (z,) * 4)
            tot = (accs[0] + accs[1]) + (accs[2] + accs[3])
            foldb[pl.ds(0, 16)] = tot
            outv[c * CS + s] = tot + foldb[pl.ds(8, 16)] + bv

    idxb, rowsb, semg, semi = bufs[0]
    stage_idx(0, idxb, semi)
    wait_idx(0, idxb, semi)
    remap(idxb)
    fire_gathers(idxb, rowsb, semg)
    stage_idx(1, bufs[1][0], bufs[1][3])
    wait_idx(1, bufs[1][0], bufs[1][3])
    remap(bufs[1][0])

    def half(c, b):
        idxb, rowsb, semg, semi = bufs[b]
        nidxb, nrowsb, nsemg, nsemi = bufs[1 - b]
        drain_gathers(rowsb, semg)

        @pl.when(c + 1 < NCHUNK)
        def _():
            fire_gathers(nidxb, nrowsb, nsemg)

        @pl.when(c + 2 < NCHUNK)
        def _():
            stage_idx(c + 2, idxb, semi)

        accumulate(c, rowsb)

        @pl.when(c + 2 < NCHUNK)
        def _():
            wait_idx(c + 2, idxb, semi)
            remap(idxb)

    def pair(cc, carry):
        half(2 * cc, 0)
        half(2 * cc + 1, 1)
        return carry

    lax.fori_loop(0, NCHUNK // 2, pair, 0)
    pltpu.sync_copy(outv, out_hbm.at[pl.ds(base, SPW)])


@jax.jit
def kernel(inputs, embed_table, W, b):
    b_pad = jnp.zeros((16,), jnp.float32).at[:NCLS].set(b)
    w_bd = jnp.zeros((SLOTS * EMBED, 128), jnp.float32)
    for _m in range(SLOTS):
        w_bd = w_bd.at[EMBED * _m:EMBED * (_m + 1),
                       DP * _m:DP * _m + NCLS].set(W)
    p = jnp.reshape(_project_table(embed_table.T, w_bd), (VP, DP))
    idx1 = jnp.reshape(inputs.astype(jnp.int32), (BATCH * HIST,))

    mesh = plsc.VectorSubcoreMesh(core_axis_name="c", subcore_axis_name="s")
    out16 = pl.kernel(
        _sc_body,
        out_type=jax.ShapeDtypeStruct((BATCH, 16), jnp.float32),
        mesh=mesh,
        compiler_params=pltpu.CompilerParams(use_tc_tiling_on_sc=False,
                                             needs_layout_passes=False),
        scratch_types=[
            pltpu.VMEM((CS * HIST,), jnp.int32),
            pltpu.VMEM((CS * HIST,), jnp.int32),
            pltpu.VMEM((CS * HIST, DP), jnp.float32),
            pltpu.VMEM((CS * HIST, DP), jnp.float32),
            pltpu.VMEM((SPW, 16), jnp.float32),
            pltpu.VMEM((16,), jnp.float32),
            pltpu.VMEM((24,), jnp.float32),
            pltpu.SemaphoreType.DMA,
            pltpu.SemaphoreType.DMA,
            pltpu.SemaphoreType.DMA,
            pltpu.SemaphoreType.DMA,
        ],
    )(p, idx1, b_pad)
    return out16[:, :NCLS]
